# baseline (speedup 1.0000x reference)
import jax
import jax.numpy as jnp
from jax.experimental import pallas as pl

_NUM_NODES = 10000
_NUM_GENE = 4000
_HID = 32
_HEADS = 2


def _seg_softmax(alpha, index, num_segments):
    amax = jax.ops.segment_max(alpha, index, num_segments=num_segments)
    amax = jnp.where(jnp.isfinite(amax), amax, 0.0)
    ex = jnp.exp(alpha - amax[index])
    denom = jax.ops.segment_sum(ex, index, num_segments=num_segments)
    return ex / (denom[index] + 1e-16)


def _head_body(xn_ref, w3_ref, b3_ref, w4_ref, b4_ref, o_ref):
    k = pl.program_id(0)
    h = jnp.dot(xn_ref[...], w3_ref[...], preferred_element_type=jnp.float32)
    h = jnp.maximum(h + b3_ref[...], 0.0)
    p = jnp.dot(h, w4_ref[...], preferred_element_type=jnp.float32)

    @pl.when(k == 0)
    def _():
        o_ref[...] = p + b4_ref[...]

    @pl.when(k != 0)
    def _():
        o_ref[...] = o_ref[...] + p


def _dense_head(xn, W3, b3, W4, b4):
    B = xn.shape[0]
    G = W3.shape[0]
    C = W4.shape[1]
    GP = ((G + 511) // 512) * 512
    W3p = jnp.pad(W3, ((0, 0), (0, GP - G)))
    b3p = jnp.pad(b3.reshape(1, G), ((0, 0), (0, GP - G)))
    W4p = jnp.pad(W4, ((0, GP - G), (0, 0)))
    KB = 512
    nk = GP // KB
    return pl.pallas_call(
        _head_body,
        grid=(nk,),
        in_specs=[
            pl.BlockSpec((B, G), lambda k: (0, 0)),
            pl.BlockSpec((G, KB), lambda k: (0, k)),
            pl.BlockSpec((1, KB), lambda k: (0, k)),
            pl.BlockSpec((KB, C), lambda k: (k, 0)),
            pl.BlockSpec((1, C), lambda k: (0, 0)),
        ],
        out_specs=pl.BlockSpec((B, C), lambda k: (0, 0)),
        out_shape=jax.ShapeDtypeStruct((B, C), jnp.float32),
    )(xn, W3p, b3p, W4p, b4.reshape(1, C))


def kernel(x, edge_index, edge_tf, batch, W1x, b1x, W1e, b1e, c1_Wl, c1_bl,
           c1_Wr, c1_br, c1_We, c1_att, c1_bias, W2, b2, c2_Wl, c2_bl, c2_Wr,
           c2_br, c2_att, c2_bias, ln_g, ln_b, W3, b3, W4, b4):
    N = x.shape[0]
    B = N // _NUM_NODES
    src, dst = edge_index[0], edge_index[1]
    x_edge = x[src] * x[dst]
    xh = jax.nn.sigmoid(x @ W1x + b1x)
    xe = jax.nn.sigmoid(x_edge @ W1e + b1e)
    xl = (xh @ c1_Wl + c1_bl).reshape(N, _HEADS, _HID)
    xr = (xh @ c1_Wr + c1_br).reshape(N, _HEADS, _HID)
    ea = (xe @ c1_We).reshape(-1, _HEADS, _HID)
    m = jax.nn.leaky_relu(xr[dst] + xl[src] + ea, 0.2)
    alpha = (m * c1_att).sum(-1)
    alpha = _seg_softmax(alpha, dst, N)
    out = jax.ops.segment_sum(xl[src] * alpha[:, :, None], dst, num_segments=N)
    out = out.reshape(N, _HEADS * _HID) + c1_bias
    x1 = out.reshape(B, _NUM_NODES, -1).mean(axis=-1)
    xt = jax.nn.sigmoid(x1.reshape(N, 1) @ W2 + b2)
    src2, dst2 = edge_tf[0], edge_tf[1]
    yl = (xt @ c2_Wl + c2_bl).reshape(N, 1, _HID)
    yr = (xt @ c2_Wr + c2_br).reshape(N, 1, _HID)
    m2 = jax.nn.leaky_relu(yr[dst2] + yl[src2], 0.2)
    a2 = (m2 * c2_att).sum(-1)
    a2 = _seg_softmax(a2, dst2, N)
    out2 = jax.ops.segment_sum(yl[src2] * a2[:, :, None], dst2, num_segments=N)
    out2 = out2.reshape(N, _HID) + c2_bias
    x2 = x1 + out2.reshape(B, _NUM_NODES, -1).mean(axis=-1)
    xg = x2[:, :_NUM_GENE]
    mu = xg.mean(axis=-1, keepdims=True)
    var = xg.var(axis=-1, keepdims=True)
    xn = (xg - mu) / jnp.sqrt(var + 1e-5) * ln_g + ln_b
    return _dense_head(xn, W3, b3, W4, b4)


# speedup vs baseline: 1.1690x; 1.1690x over previous
import functools

import jax
import jax.numpy as jnp
from jax import lax
from jax.experimental import pallas as pl
from jax.experimental.pallas import tpu as pltpu
from jax.experimental.pallas import tpu_sc as plsc

_NUM_NODES = 10000
_NUM_GENE = 4000
_HID = 32
_HEADS = 2

_NC = 2
_NS = 16
_NW = _NC * _NS
_L = 16
_EB = 128
_PW = 8


def _seg_softmax(alpha, index, num_segments):
    amax = jax.ops.segment_max(alpha, index, num_segments=num_segments)
    amax = jnp.where(jnp.isfinite(amax), amax, 0.0)
    ex = jnp.exp(alpha - amax[index])
    denom = jax.ops.segment_sum(ex, index, num_segments=num_segments)
    return ex / (denom[index] + 1e-16)


def _sc_l2_body(src_hbm, dst_hbm, yl_hbm, yr_hbm, att_hbm, zero_hbm, out_hbm,
                idx_s, idx_d, ylv, yrv, pay, att_v, acc, sem1, sem2):
    cid = lax.axis_index("c")
    sid = lax.axis_index("s")
    wid = sid * _NC + cid
    nchunk = src_hbm.shape[0] // (_NW * _EB)
    iot = lax.iota(jnp.int32, _L)

    @pl.when(sid == 0)
    def _():
        pltpu.sync_copy(zero_hbm, acc)

    pltpu.sync_copy(att_hbm, att_v)
    plsc.subcore_barrier()

    def chunk(g, carry):
        base = (wid * nchunk + g) * _EB
        pltpu.sync_copy(src_hbm.at[pl.ds(base, _EB)], idx_s)
        pltpu.sync_copy(dst_hbm.at[pl.ds(base, _EB)], idx_d)
        cp1 = pltpu.async_copy(yl_hbm.at[idx_s], ylv, sem1)
        cp2 = pltpu.async_copy(yr_hbm.at[idx_d], yrv, sem2)
        cp1.wait()
        cp2.wait()

        def group(j, carry2):
            rows = j * _L + iot
            col0 = jnp.zeros((_L,), jnp.int32)

            def ch(c, ac):
                a, sl = ac
                cc = col0 + c
                ylc = plsc.load_gather(ylv, [rows, cc])
                yrc = plsc.load_gather(yrv, [rows, cc])
                attc = plsc.load_gather(att_v, [cc])
                z = ylc + yrc
                m = jnp.maximum(z, 0.2 * z)
                return (a + m * attc, sl + ylc)

            a, sl = lax.fori_loop(
                0, _HID, ch,
                (jnp.zeros((_L,), jnp.float32), jnp.zeros((_L,), jnp.float32)))
            ex = jnp.exp(a)
            plsc.store_scatter(pay, [rows, col0], ex * sl)
            plsc.store_scatter(pay, [rows, col0 + 1], ex)
            return carry2

        lax.fori_loop(0, _EB // _L, group, 0)
        pltpu.sync_copy(pay, acc.at[idx_d], add=True)
        return carry

    lax.fori_loop(0, nchunk, chunk, 0)
    plsc.subcore_barrier()

    @pl.when(sid == 0)
    def _():
        pltpu.sync_copy(acc, out_hbm.at[cid])


def _sc_l2(src2p, dst2p, yl, yr, att, n_acc):
    mesh = plsc.VectorSubcoreMesh(core_axis_name="c", subcore_axis_name="s")
    zero = jnp.zeros((n_acc, _PW), jnp.float32)
    fn = pl.kernel(
        _sc_l2_body,
        out_type=jax.ShapeDtypeStruct((_NC, n_acc, _PW), jnp.float32),
        mesh=mesh,
        scratch_types=[
            pltpu.VMEM((_EB,), jnp.int32),
            pltpu.VMEM((_EB,), jnp.int32),
            pltpu.VMEM((_EB, _HID), jnp.float32),
            pltpu.VMEM((_EB, _HID), jnp.float32),
            pltpu.VMEM((_EB, _PW), jnp.float32),
            pltpu.VMEM((_HID,), jnp.float32),
            pltpu.VMEM_SHARED((n_acc, _PW), jnp.float32),
            pltpu.SemaphoreType.DMA,
            pltpu.SemaphoreType.DMA,
        ],
        compiler_params=pltpu.CompilerParams(
            needs_layout_passes=False, use_tc_tiling_on_sc=False),
    )
    return fn(src2p, dst2p, yl, yr, att, zero)


def _head_body(xn_ref, w3_ref, b3_ref, w4_ref, b4_ref, o_ref):
    k = pl.program_id(0)
    h = jnp.dot(xn_ref[...], w3_ref[...], preferred_element_type=jnp.float32)
    h = jnp.maximum(h + b3_ref[...], 0.0)
    p = jnp.dot(h, w4_ref[...], preferred_element_type=jnp.float32)

    @pl.when(k == 0)
    def _():
        o_ref[...] = p + b4_ref[...]

    @pl.when(k != 0)
    def _():
        o_ref[...] = o_ref[...] + p


def _dense_head(xn, W3, b3, W4, b4):
    B = xn.shape[0]
    G = W3.shape[0]
    C = W4.shape[1]
    GP = ((G + 511) // 512) * 512
    W3p = jnp.pad(W3, ((0, 0), (0, GP - G)))
    b3p = jnp.pad(b3.reshape(1, G), ((0, 0), (0, GP - G)))
    W4p = jnp.pad(W4, ((0, GP - G), (0, 0)))
    KB = 512
    nk = GP // KB
    return pl.pallas_call(
        _head_body,
        grid=(nk,),
        in_specs=[
            pl.BlockSpec((B, G), lambda k: (0, 0)),
            pl.BlockSpec((G, KB), lambda k: (0, k)),
            pl.BlockSpec((1, KB), lambda k: (0, k)),
            pl.BlockSpec((KB, C), lambda k: (k, 0)),
            pl.BlockSpec((1, C), lambda k: (0, 0)),
        ],
        out_specs=pl.BlockSpec((B, C), lambda k: (0, 0)),
        out_shape=jax.ShapeDtypeStruct((B, C), jnp.float32),
    )(xn, W3p, b3p, W4p, b4.reshape(1, C))


def kernel(x, edge_index, edge_tf, batch, W1x, b1x, W1e, b1e, c1_Wl, c1_bl,
           c1_Wr, c1_br, c1_We, c1_att, c1_bias, W2, b2, c2_Wl, c2_bl, c2_Wr,
           c2_br, c2_att, c2_bias, ln_g, ln_b, W3, b3, W4, b4):
    N = x.shape[0]
    B = N // _NUM_NODES

    src, dst = edge_index[0], edge_index[1]
    x_edge = x[src] * x[dst]
    xh = jax.nn.sigmoid(x @ W1x + b1x)
    xe = jax.nn.sigmoid(x_edge @ W1e + b1e)
    xl = (xh @ c1_Wl + c1_bl).reshape(N, _HEADS, _HID)
    xr = (xh @ c1_Wr + c1_br).reshape(N, _HEADS, _HID)
    ea = (xe @ c1_We).reshape(-1, _HEADS, _HID)
    m = jax.nn.leaky_relu(xr[dst] + xl[src] + ea, 0.2)
    alpha = (m * c1_att).sum(-1)
    alpha = _seg_softmax(alpha, dst, N)
    out = jax.ops.segment_sum(xl[src] * alpha[:, :, None], dst, num_segments=N)
    out = out.reshape(N, _HEADS * _HID) + c1_bias
    x1 = out.reshape(B, _NUM_NODES, -1).mean(axis=-1)

    xt = jax.nn.sigmoid(x1.reshape(N, 1) @ W2 + b2)
    yl2 = xt @ c2_Wl + c2_bl
    yr2 = xt @ c2_Wr + c2_br

    E2 = edge_tf.shape[1]
    e2p = ((E2 + _NW * _EB - 1) // (_NW * _EB)) * (_NW * _EB)
    n_acc = ((N + 1 + 15) // 16) * 16
    src2p = jnp.pad(edge_tf[0], (0, e2p - E2))
    dst2p = jnp.pad(edge_tf[1], (0, e2p - E2), constant_values=N)
    att2 = c2_att.reshape(_HID)

    parts = _sc_l2(src2p, dst2p, yl2, yr2, att2, n_acc)
    nd = parts[0] + parts[1]
    num2 = nd[:N, 0]
    den2 = nd[:N, 1]
    out2_mean = num2 / (den2 + 1e-16) / _HID + jnp.mean(c2_bias)
    x2 = x1 + out2_mean.reshape(B, _NUM_NODES)

    xg = x2[:, :_NUM_GENE]
    mu = xg.mean(axis=-1, keepdims=True)
    var = xg.var(axis=-1, keepdims=True)
    xn = (xg - mu) / jnp.sqrt(var + 1e-5) * ln_g + ln_b
    return _dense_head(xn, W3, b3, W4, b4)


# speedup vs baseline: 25.6614x; 21.9522x over previous
import functools

import jax
import jax.numpy as jnp
from jax import lax
from jax.experimental import pallas as pl
from jax.experimental.pallas import tpu as pltpu
from jax.experimental.pallas import tpu_sc as plsc

_NUM_NODES = 10000
_NUM_GENE = 4000
_HID = 32
_HEADS = 2

_NC = 2
_NS = 16
_NW = _NC * _NS
_L = 16
_EB = 128
_PW = 8


_EBO = 512
_NSUB = _EBO // _EB


def _sc_l1_gather_body(src_hbm, dst_hbm, x_hbm, xl_hbm, xr_hbm,
                       xs_out, xd_out, xlg_out, xrg_out,
                       idx_s, idx_d, xsv, xdv, xlv, xrv, sem):
    cid = lax.axis_index("c")
    sid = lax.axis_index("s")
    wid = sid * _NC + cid
    nchunk = src_hbm.shape[0] // (_NW * _EBO)

    def chunk(g, carry):
        base = (wid * nchunk + g) * _EBO
        pltpu.sync_copy(src_hbm.at[pl.ds(base, _EBO)], idx_s)
        pltpu.sync_copy(dst_hbm.at[pl.ds(base, _EBO)], idx_d)
        cps = []
        for q in range(_NSUB):
            r = pl.ds(q * _EB, _EB)
            cps.append(pltpu.async_copy(x_hbm.at[idx_s.at[r]], xsv.at[r], sem))
            cps.append(pltpu.async_copy(x_hbm.at[idx_d.at[r]], xdv.at[r], sem))
            cps.append(pltpu.async_copy(xl_hbm.at[idx_s.at[r]], xlv.at[r], sem))
            cps.append(pltpu.async_copy(xr_hbm.at[idx_d.at[r]], xrv.at[r], sem))
        for cp in cps:
            cp.wait()
        pltpu.sync_copy(xsv, xs_out.at[pl.ds(base, _EBO)])
        pltpu.sync_copy(xdv, xd_out.at[pl.ds(base, _EBO)])
        pltpu.sync_copy(xlv, xlg_out.at[pl.ds(base, _EBO)])
        pltpu.sync_copy(xrv, xrg_out.at[pl.ds(base, _EBO)])
        return carry

    lax.fori_loop(0, nchunk, chunk, 0)


def _sc_l1_gather(srcp, dstp, x, xl, xr):
    e1p = srcp.shape[0]
    mesh = plsc.VectorSubcoreMesh(core_axis_name="c", subcore_axis_name="s")
    fn = pl.kernel(
        _sc_l1_gather_body,
        out_type=(
            jax.ShapeDtypeStruct((e1p, 16), jnp.float32),
            jax.ShapeDtypeStruct((e1p, 16), jnp.float32),
            jax.ShapeDtypeStruct((e1p, 64), jnp.float32),
            jax.ShapeDtypeStruct((e1p, 64), jnp.float32),
        ),
        mesh=mesh,
        scratch_types=[
            pltpu.VMEM((_EBO,), jnp.int32),
            pltpu.VMEM((_EBO,), jnp.int32),
            pltpu.VMEM((_EBO, 16), jnp.float32),
            pltpu.VMEM((_EBO, 16), jnp.float32),
            pltpu.VMEM((_EBO, 64), jnp.float32),
            pltpu.VMEM((_EBO, 64), jnp.float32),
            pltpu.SemaphoreType.DMA,
        ],
        compiler_params=pltpu.CompilerParams(
            needs_layout_passes=False, use_tc_tiling_on_sc=False),
    )
    return fn(srcp, dstp, x, xl, xr)


def _alpha_body(xs_ref, xd_ref, xlg_ref, xrg_ref, w1e_ref, b1e_ref, we_ref,
                att_ref, sel_ref, ex_ref):
    u = xs_ref[...] * xd_ref[...]
    xe = jax.nn.sigmoid(
        jnp.dot(u, w1e_ref[...], preferred_element_type=jnp.float32)
        + b1e_ref[...])
    ea = jnp.dot(xe, we_ref[...], preferred_element_type=jnp.float32)
    mm = xlg_ref[...] + xrg_ref[...] + ea
    m = jnp.maximum(mm, 0.2 * mm)
    aw = m * att_ref[...]
    alpha = jnp.dot(aw, sel_ref[...], preferred_element_type=jnp.float32)
    ex_ref[...] = jnp.exp(alpha)


def _tc_alpha(xs, xd, xlg, xrg, W1e, b1e, c1_We, c1_att):
    e1p = xs.shape[0]
    EBK = 4096
    nk = e1p // EBK
    attv = c1_att.reshape(1, _HEADS * _HID)
    sel = jnp.repeat(jnp.eye(_HEADS, dtype=jnp.float32), _HID, axis=0)
    return pl.pallas_call(
        _alpha_body,
        grid=(nk,),
        in_specs=[
            pl.BlockSpec((EBK, 16), lambda k: (k, 0)),
            pl.BlockSpec((EBK, 16), lambda k: (k, 0)),
            pl.BlockSpec((EBK, 64), lambda k: (k, 0)),
            pl.BlockSpec((EBK, 64), lambda k: (k, 0)),
            pl.BlockSpec((16, 32), lambda k: (0, 0)),
            pl.BlockSpec((1, 32), lambda k: (0, 0)),
            pl.BlockSpec((32, 64), lambda k: (0, 0)),
            pl.BlockSpec((1, 64), lambda k: (0, 0)),
            pl.BlockSpec((64, _HEADS), lambda k: (0, 0)),
        ],
        out_specs=pl.BlockSpec((EBK, _HEADS), lambda k: (k, 0)),
        out_shape=jax.ShapeDtypeStruct((e1p, _HEADS), jnp.float32),
    )(xs, xd, xlg, xrg, W1e, b1e.reshape(1, 32), c1_We, attv, sel)


def _sc_l1_scatter_body(src_hbm, dst_hbm, ex_hbm, sl_hbm, zero_hbm, out_hbm,
                        idx_s, idx_d2, exv, slv, pay, acc, sem):
    cid = lax.axis_index("c")
    sid = lax.axis_index("s")
    wid = sid * _NC + cid
    nchunk = src_hbm.shape[0] // (_NW * _EBO)
    iot = lax.iota(jnp.int32, _L)

    @pl.when(sid == 0)
    def _():
        pltpu.sync_copy(zero_hbm, acc)

    def zinit(j, carry):
        rows = j * _L + iot
        zv = jnp.zeros((_L,), jnp.float32)
        c4 = jnp.zeros((_L,), jnp.int32) + 4
        for q in range(4):
            plsc.store_scatter(pay, [rows, c4 + q], zv)
        return carry

    lax.fori_loop(0, _EBO // _L, zinit, 0)
    plsc.subcore_barrier()

    def chunk(g, carry):
        base = (wid * nchunk + g) * _EBO
        pltpu.sync_copy(src_hbm.at[pl.ds(base, _EBO)], idx_s)
        for q in range(_NSUB):
            pltpu.sync_copy(dst_hbm.at[pl.ds(base + q * _EB, _EB)],
                            idx_d2.at[q])
        pltpu.sync_copy(ex_hbm.at[pl.ds(base, _EBO)], exv)
        cps = []
        for q in range(_NSUB):
            r = pl.ds(q * _EB, _EB)
            cps.append(pltpu.async_copy(sl_hbm.at[idx_s.at[r]], slv.at[r], sem))
        for cp in cps:
            cp.wait()

        def group(j, carry2):
            rows = j * _L + iot
            col0 = jnp.zeros((_L,), jnp.int32)
            ex0 = plsc.load_gather(exv, [rows, col0])
            ex1 = plsc.load_gather(exv, [rows, col0 + 1])
            sl0 = plsc.load_gather(slv, [rows, col0])
            sl1 = plsc.load_gather(slv, [rows, col0 + 1])
            plsc.store_scatter(pay, [rows, col0], ex0 * sl0)
            plsc.store_scatter(pay, [rows, col0 + 1], ex1 * sl1)
            plsc.store_scatter(pay, [rows, col0 + 2], ex0)
            plsc.store_scatter(pay, [rows, col0 + 3], ex1)
            return carry2

        lax.fori_loop(0, _EBO // _L, group, 0)
        for q in range(_NSUB):
            pltpu.sync_copy(pay.at[pl.ds(q * _EB, _EB)],
                            acc.at[idx_d2.at[q]], add=True)
        return carry

    lax.fori_loop(0, nchunk, chunk, 0)
    plsc.subcore_barrier()

    @pl.when(sid == 0)
    def _():
        pltpu.sync_copy(acc, out_hbm.at[cid])


def _sc_l1_scatter(srcp, dstp, ex, sl16, n_acc):
    e1p = srcp.shape[0]
    mesh = plsc.VectorSubcoreMesh(core_axis_name="c", subcore_axis_name="s")
    zero = jnp.zeros((n_acc, _PW), jnp.float32)
    fn = pl.kernel(
        _sc_l1_scatter_body,
        out_type=jax.ShapeDtypeStruct((_NC, n_acc, _PW), jnp.float32),
        mesh=mesh,
        scratch_types=[
            pltpu.VMEM((_EBO,), jnp.int32),
            pltpu.VMEM((_NSUB, _EB), jnp.int32),
            pltpu.VMEM((_EBO, _HEADS), jnp.float32),
            pltpu.VMEM((_EBO, 16), jnp.float32),
            pltpu.VMEM((_EBO, _PW), jnp.float32),
            pltpu.VMEM_SHARED((n_acc, _PW), jnp.float32),
            pltpu.SemaphoreType.DMA,
        ],
        compiler_params=pltpu.CompilerParams(
            needs_layout_passes=False, use_tc_tiling_on_sc=False),
    )
    return fn(srcp, dstp, ex, sl16, zero)


def _sc_l2_body(src_hbm, dst_hbm, yl_hbm, yr_hbm, att_hbm, zero_hbm, out_hbm,
                idx_s, idx_d, ylv, yrv, pay, att_v, acc, sem1, sem2):
    cid = lax.axis_index("c")
    sid = lax.axis_index("s")
    wid = sid * _NC + cid
    nchunk = src_hbm.shape[0] // (_NW * _EB)
    iot = lax.iota(jnp.int32, _L)

    @pl.when(sid == 0)
    def _():
        pltpu.sync_copy(zero_hbm, acc)

    pltpu.sync_copy(att_hbm, att_v)
    plsc.subcore_barrier()

    def chunk(g, carry):
        base = (wid * nchunk + g) * _EB
        pltpu.sync_copy(src_hbm.at[pl.ds(base, _EB)], idx_s)
        pltpu.sync_copy(dst_hbm.at[pl.ds(base, _EB)], idx_d)
        cp1 = pltpu.async_copy(yl_hbm.at[idx_s], ylv, sem1)
        cp2 = pltpu.async_copy(yr_hbm.at[idx_d], yrv, sem2)
        cp1.wait()
        cp2.wait()

        def group(j, carry2):
            rows = j * _L + iot
            col0 = jnp.zeros((_L,), jnp.int32)

            def ch(c, ac):
                a, sl = ac
                cc = col0 + c
                ylc = plsc.load_gather(ylv, [rows, cc])
                yrc = plsc.load_gather(yrv, [rows, cc])
                attc = plsc.load_gather(att_v, [cc])
                z = ylc + yrc
                m = jnp.maximum(z, 0.2 * z)
                return (a + m * attc, sl + ylc)

            a, sl = lax.fori_loop(
                0, _HID, ch,
                (jnp.zeros((_L,), jnp.float32), jnp.zeros((_L,), jnp.float32)))
            ex = jnp.exp(a)
            plsc.store_scatter(pay, [rows, col0], ex * sl)
            plsc.store_scatter(pay, [rows, col0 + 1], ex)
            return carry2

        lax.fori_loop(0, _EB // _L, group, 0)
        pltpu.sync_copy(pay, acc.at[idx_d], add=True)
        return carry

    lax.fori_loop(0, nchunk, chunk, 0)
    plsc.subcore_barrier()

    @pl.when(sid == 0)
    def _():
        pltpu.sync_copy(acc, out_hbm.at[cid])


def _sc_l2(src2p, dst2p, yl, yr, att, n_acc):
    mesh = plsc.VectorSubcoreMesh(core_axis_name="c", subcore_axis_name="s")
    zero = jnp.zeros((n_acc, _PW), jnp.float32)
    fn = pl.kernel(
        _sc_l2_body,
        out_type=jax.ShapeDtypeStruct((_NC, n_acc, _PW), jnp.float32),
        mesh=mesh,
        scratch_types=[
            pltpu.VMEM((_EB,), jnp.int32),
            pltpu.VMEM((_EB,), jnp.int32),
            pltpu.VMEM((_EB, _HID), jnp.float32),
            pltpu.VMEM((_EB, _HID), jnp.float32),
            pltpu.VMEM((_EB, _PW), jnp.float32),
            pltpu.VMEM((_HID,), jnp.float32),
            pltpu.VMEM_SHARED((n_acc, _PW), jnp.float32),
            pltpu.SemaphoreType.DMA,
            pltpu.SemaphoreType.DMA,
        ],
        compiler_params=pltpu.CompilerParams(
            needs_layout_passes=False, use_tc_tiling_on_sc=False),
    )
    return fn(src2p, dst2p, yl, yr, att, zero)


def _head_body(xn_ref, w3_ref, b3_ref, w4_ref, b4_ref, o_ref):
    k = pl.program_id(0)
    h = jnp.dot(xn_ref[...], w3_ref[...], preferred_element_type=jnp.float32)
    h = jnp.maximum(h + b3_ref[...], 0.0)
    p = jnp.dot(h, w4_ref[...], preferred_element_type=jnp.float32)

    @pl.when(k == 0)
    def _():
        o_ref[...] = p + b4_ref[...]

    @pl.when(k != 0)
    def _():
        o_ref[...] = o_ref[...] + p


def _dense_head(xn, W3, b3, W4, b4):
    B = xn.shape[0]
    G = W3.shape[0]
    C = W4.shape[1]
    GP = ((G + 511) // 512) * 512
    W3p = jnp.pad(W3, ((0, 0), (0, GP - G)))
    b3p = jnp.pad(b3.reshape(1, G), ((0, 0), (0, GP - G)))
    W4p = jnp.pad(W4, ((0, GP - G), (0, 0)))
    KB = 512
    nk = GP // KB
    return pl.pallas_call(
        _head_body,
        grid=(nk,),
        in_specs=[
            pl.BlockSpec((B, G), lambda k: (0, 0)),
            pl.BlockSpec((G, KB), lambda k: (0, k)),
            pl.BlockSpec((1, KB), lambda k: (0, k)),
            pl.BlockSpec((KB, C), lambda k: (k, 0)),
            pl.BlockSpec((1, C), lambda k: (0, 0)),
        ],
        out_specs=pl.BlockSpec((B, C), lambda k: (0, 0)),
        out_shape=jax.ShapeDtypeStruct((B, C), jnp.float32),
    )(xn, W3p, b3p, W4p, b4.reshape(1, C))


def kernel(x, edge_index, edge_tf, batch, W1x, b1x, W1e, b1e, c1_Wl, c1_bl,
           c1_Wr, c1_br, c1_We, c1_att, c1_bias, W2, b2, c2_Wl, c2_bl, c2_Wr,
           c2_br, c2_att, c2_bias, ln_g, ln_b, W3, b3, W4, b4):
    N = x.shape[0]
    B = N // _NUM_NODES
    n_acc = ((N + 1 + 15) // 16) * 16

    E1 = edge_index.shape[1]
    e1p = ((E1 + _NW * _EBO - 1) // (_NW * _EBO)) * (_NW * _EBO)
    srcp = jnp.pad(edge_index[0], (0, e1p - E1))
    dstp = jnp.pad(edge_index[1], (0, e1p - E1), constant_values=N)

    xh = jax.nn.sigmoid(x @ W1x + b1x)
    xl = xh @ c1_Wl + c1_bl
    xr = xh @ c1_Wr + c1_br
    sl1 = xl.reshape(N, _HEADS, _HID).sum(-1)
    sl116 = jnp.zeros((N, 16), jnp.float32).at[:, :_HEADS].set(sl1)

    xs, xd, xlg, xrg = _sc_l1_gather(srcp, dstp, x, xl, xr)
    ex1 = _tc_alpha(xs, xd, xlg, xrg, W1e, b1e, c1_We, c1_att)
    parts1 = _sc_l1_scatter(srcp, dstp, ex1, sl116, n_acc)
    nd1 = parts1[0] + parts1[1]
    x1 = ((nd1[:N, 0] / (nd1[:N, 2] + 1e-16)
           + nd1[:N, 1] / (nd1[:N, 3] + 1e-16)) / (_HEADS * _HID)
          + jnp.mean(c1_bias)).reshape(B, _NUM_NODES)

    xt = jax.nn.sigmoid(x1.reshape(N, 1) @ W2 + b2)
    yl2 = xt @ c2_Wl + c2_bl
    yr2 = xt @ c2_Wr + c2_br

    E2 = edge_tf.shape[1]
    e2p = ((E2 + _NW * _EB - 1) // (_NW * _EB)) * (_NW * _EB)
    src2p = jnp.pad(edge_tf[0], (0, e2p - E2))
    dst2p = jnp.pad(edge_tf[1], (0, e2p - E2), constant_values=N)
    att2 = c2_att.reshape(_HID)

    parts = _sc_l2(src2p, dst2p, yl2, yr2, att2, n_acc)
    nd = parts[0] + parts[1]
    num2 = nd[:N, 0]
    den2 = nd[:N, 1]
    out2_mean = num2 / (den2 + 1e-16) / _HID + jnp.mean(c2_bias)
    x2 = x1 + out2_mean.reshape(B, _NUM_NODES)

    xg = x2[:, :_NUM_GENE]
    mu = xg.mean(axis=-1, keepdims=True)
    var = xg.var(axis=-1, keepdims=True)
    xn = (xg - mu) / jnp.sqrt(var + 1e-5) * ln_g + ln_b
    return _dense_head(xn, W3, b3, W4, b4)


# speedup vs baseline: 28.2116x; 1.0994x over previous
import functools

import jax
import jax.numpy as jnp
from jax import lax
from jax.experimental import pallas as pl
from jax.experimental.pallas import tpu as pltpu
from jax.experimental.pallas import tpu_sc as plsc

_NUM_NODES = 10000
_NUM_GENE = 4000
_HID = 32
_HEADS = 2

_NC = 2
_NS = 16
_NW = _NC * _NS
_L = 16
_EB = 128
_PW = 8


_EBO = 512
_NSUB = _EBO // _EB


def _sc_l1_gather_body(src_hbm, dst_hbm, x_hbm, xl_hbm, xr_hbm,
                       xs_out, xd_out, xlg_out, xrg_out,
                       idx_s, idx_d, xsv, xdv, xlv, xrv, sem):
    cid = lax.axis_index("c")
    sid = lax.axis_index("s")
    wid = sid * _NC + cid
    nchunk = src_hbm.shape[0] // (_NW * _EBO)

    def chunk(g, carry):
        base = (wid * nchunk + g) * _EBO
        pltpu.sync_copy(src_hbm.at[pl.ds(base, _EBO)], idx_s)
        pltpu.sync_copy(dst_hbm.at[pl.ds(base, _EBO)], idx_d)
        cps = []
        for q in range(_NSUB):
            r = pl.ds(q * _EB, _EB)
            cps.append(pltpu.async_copy(x_hbm.at[idx_s.at[r]], xsv.at[r], sem))
            cps.append(pltpu.async_copy(x_hbm.at[idx_d.at[r]], xdv.at[r], sem))
            cps.append(pltpu.async_copy(xl_hbm.at[idx_s.at[r]], xlv.at[r], sem))
            cps.append(pltpu.async_copy(xr_hbm.at[idx_d.at[r]], xrv.at[r], sem))
        for cp in cps:
            cp.wait()
        pltpu.sync_copy(xsv, xs_out.at[pl.ds(base, _EBO)])
        pltpu.sync_copy(xdv, xd_out.at[pl.ds(base, _EBO)])
        pltpu.sync_copy(xlv, xlg_out.at[pl.ds(base, _EBO)])
        pltpu.sync_copy(xrv, xrg_out.at[pl.ds(base, _EBO)])
        return carry

    lax.fori_loop(0, nchunk, chunk, 0)


def _sc_l1_gather(srcp, dstp, x, xl, xr):
    e1p = srcp.shape[0]
    mesh = plsc.VectorSubcoreMesh(core_axis_name="c", subcore_axis_name="s")
    fn = pl.kernel(
        _sc_l1_gather_body,
        out_type=(
            jax.ShapeDtypeStruct((e1p, 16), jnp.float32),
            jax.ShapeDtypeStruct((e1p, 16), jnp.float32),
            jax.ShapeDtypeStruct((e1p, 64), jnp.float32),
            jax.ShapeDtypeStruct((e1p, 64), jnp.float32),
        ),
        mesh=mesh,
        scratch_types=[
            pltpu.VMEM((_EBO,), jnp.int32),
            pltpu.VMEM((_EBO,), jnp.int32),
            pltpu.VMEM((_EBO, 16), jnp.float32),
            pltpu.VMEM((_EBO, 16), jnp.float32),
            pltpu.VMEM((_EBO, 64), jnp.float32),
            pltpu.VMEM((_EBO, 64), jnp.float32),
            pltpu.SemaphoreType.DMA,
        ],
        compiler_params=pltpu.CompilerParams(
            needs_layout_passes=False, use_tc_tiling_on_sc=False),
    )
    return fn(srcp, dstp, x, xl, xr)


def _alpha_body(xs_ref, xd_ref, xlg_ref, xrg_ref, w1e_ref, b1e_ref, we_ref,
                att_ref, sel_ref, pay_ref):
    u = xs_ref[...] * xd_ref[...]
    xe = jax.nn.sigmoid(
        jnp.dot(u, w1e_ref[...], preferred_element_type=jnp.float32)
        + b1e_ref[...])
    ea = jnp.dot(xe, we_ref[...], preferred_element_type=jnp.float32)
    mm = xlg_ref[...] + xrg_ref[...] + ea
    m = jnp.maximum(mm, 0.2 * mm)
    aw = m * att_ref[...]
    alpha = jnp.dot(aw, sel_ref[...], preferred_element_type=jnp.float32)
    ex = jnp.exp(alpha)
    sl = jnp.dot(xlg_ref[...], sel_ref[...],
                 preferred_element_type=jnp.float32)
    z = jnp.zeros_like(ex)
    pay_ref[...] = jnp.concatenate([ex * sl, ex, z, z], axis=-1)


def _tc_alpha(xs, xd, xlg, xrg, W1e, b1e, c1_We, c1_att):
    e1p = xs.shape[0]
    EBK = 4096
    nk = e1p // EBK
    attv = c1_att.reshape(1, _HEADS * _HID)
    sel = jnp.repeat(jnp.eye(_HEADS, dtype=jnp.float32), _HID, axis=0)
    return pl.pallas_call(
        _alpha_body,
        grid=(nk,),
        in_specs=[
            pl.BlockSpec((EBK, 16), lambda k: (k, 0)),
            pl.BlockSpec((EBK, 16), lambda k: (k, 0)),
            pl.BlockSpec((EBK, 64), lambda k: (k, 0)),
            pl.BlockSpec((EBK, 64), lambda k: (k, 0)),
            pl.BlockSpec((16, 32), lambda k: (0, 0)),
            pl.BlockSpec((1, 32), lambda k: (0, 0)),
            pl.BlockSpec((32, 64), lambda k: (0, 0)),
            pl.BlockSpec((1, 64), lambda k: (0, 0)),
            pl.BlockSpec((64, _HEADS), lambda k: (0, 0)),
        ],
        out_specs=pl.BlockSpec((EBK, _PW), lambda k: (k, 0)),
        out_shape=jax.ShapeDtypeStruct((e1p, _PW), jnp.float32),
    )(xs, xd, xlg, xrg, W1e, b1e.reshape(1, 32), c1_We, attv, sel)


def _sc_l1_scatter_body(dst_hbm, pay_hbm, zero_hbm, out_hbm,
                        idx_d2, pay, acc):
    cid = lax.axis_index("c")
    sid = lax.axis_index("s")
    wid = sid * _NC + cid
    nchunk = dst_hbm.shape[0] // (_NW * _EBO)

    @pl.when(sid == 0)
    def _():
        pltpu.sync_copy(zero_hbm, acc)

    plsc.subcore_barrier()

    def chunk(g, carry):
        base = (wid * nchunk + g) * _EBO
        for q in range(_NSUB):
            pltpu.sync_copy(dst_hbm.at[pl.ds(base + q * _EB, _EB)],
                            idx_d2.at[q])
        pltpu.sync_copy(pay_hbm.at[pl.ds(base, _EBO)], pay)
        for q in range(_NSUB):
            pltpu.sync_copy(pay.at[pl.ds(q * _EB, _EB)],
                            acc.at[idx_d2.at[q]], add=True)
        return carry

    lax.fori_loop(0, nchunk, chunk, 0)
    plsc.subcore_barrier()

    @pl.when(sid == 0)
    def _():
        pltpu.sync_copy(acc, out_hbm.at[cid])


def _sc_l1_scatter(dstp, pay, n_acc):
    mesh = plsc.VectorSubcoreMesh(core_axis_name="c", subcore_axis_name="s")
    zero = jnp.zeros((n_acc, _PW), jnp.float32)
    fn = pl.kernel(
        _sc_l1_scatter_body,
        out_type=jax.ShapeDtypeStruct((_NC, n_acc, _PW), jnp.float32),
        mesh=mesh,
        scratch_types=[
            pltpu.VMEM((_NSUB, _EB), jnp.int32),
            pltpu.VMEM((_EBO, _PW), jnp.float32),
            pltpu.VMEM_SHARED((n_acc, _PW), jnp.float32),
        ],
        compiler_params=pltpu.CompilerParams(
            needs_layout_passes=False, use_tc_tiling_on_sc=False),
    )
    return fn(dstp, pay, zero)


def _sc_l2_body(src_hbm, dst_hbm, yl_hbm, yr_hbm, att_hbm, zero_hbm, out_hbm,
                idx_s, idx_d, ylv, yrv, pay, att_v, acc, sem1, sem2):
    cid = lax.axis_index("c")
    sid = lax.axis_index("s")
    wid = sid * _NC + cid
    nchunk = src_hbm.shape[0] // (_NW * _EB)
    iot = lax.iota(jnp.int32, _L)

    @pl.when(sid == 0)
    def _():
        pltpu.sync_copy(zero_hbm, acc)

    pltpu.sync_copy(att_hbm, att_v)
    plsc.subcore_barrier()

    def chunk(g, carry):
        base = (wid * nchunk + g) * _EB
        pltpu.sync_copy(src_hbm.at[pl.ds(base, _EB)], idx_s)
        pltpu.sync_copy(dst_hbm.at[pl.ds(base, _EB)], idx_d)
        cp1 = pltpu.async_copy(yl_hbm.at[idx_s], ylv, sem1)
        cp2 = pltpu.async_copy(yr_hbm.at[idx_d], yrv, sem2)
        cp1.wait()
        cp2.wait()

        def group(j, carry2):
            rows = j * _L + iot
            col0 = jnp.zeros((_L,), jnp.int32)

            def ch(c, ac):
                a, sl = ac
                cc = col0 + c
                ylc = plsc.load_gather(ylv, [rows, cc])
                yrc = plsc.load_gather(yrv, [rows, cc])
                attc = plsc.load_gather(att_v, [cc])
                z = ylc + yrc
                m = jnp.maximum(z, 0.2 * z)
                return (a + m * attc, sl + ylc)

            a, sl = lax.fori_loop(
                0, _HID, ch,
                (jnp.zeros((_L,), jnp.float32), jnp.zeros((_L,), jnp.float32)))
            ex = jnp.exp(a)
            plsc.store_scatter(pay, [rows, col0], ex * sl)
            plsc.store_scatter(pay, [rows, col0 + 1], ex)
            return carry2

        lax.fori_loop(0, _EB // _L, group, 0)
        pltpu.sync_copy(pay, acc.at[idx_d], add=True)
        return carry

    lax.fori_loop(0, nchunk, chunk, 0)
    plsc.subcore_barrier()

    @pl.when(sid == 0)
    def _():
        pltpu.sync_copy(acc, out_hbm.at[cid])


def _sc_l2(src2p, dst2p, yl, yr, att, n_acc):
    mesh = plsc.VectorSubcoreMesh(core_axis_name="c", subcore_axis_name="s")
    zero = jnp.zeros((n_acc, _PW), jnp.float32)
    fn = pl.kernel(
        _sc_l2_body,
        out_type=jax.ShapeDtypeStruct((_NC, n_acc, _PW), jnp.float32),
        mesh=mesh,
        scratch_types=[
            pltpu.VMEM((_EB,), jnp.int32),
            pltpu.VMEM((_EB,), jnp.int32),
            pltpu.VMEM((_EB, _HID), jnp.float32),
            pltpu.VMEM((_EB, _HID), jnp.float32),
            pltpu.VMEM((_EB, _PW), jnp.float32),
            pltpu.VMEM((_HID,), jnp.float32),
            pltpu.VMEM_SHARED((n_acc, _PW), jnp.float32),
            pltpu.SemaphoreType.DMA,
            pltpu.SemaphoreType.DMA,
        ],
        compiler_params=pltpu.CompilerParams(
            needs_layout_passes=False, use_tc_tiling_on_sc=False),
    )
    return fn(src2p, dst2p, yl, yr, att, zero)


def _head_body(xn_ref, w3_ref, b3_ref, w4_ref, b4_ref, o_ref):
    k = pl.program_id(0)
    h = jnp.dot(xn_ref[...], w3_ref[...], preferred_element_type=jnp.float32)
    h = jnp.maximum(h + b3_ref[...], 0.0)
    p = jnp.dot(h, w4_ref[...], preferred_element_type=jnp.float32)

    @pl.when(k == 0)
    def _():
        o_ref[...] = p + b4_ref[...]

    @pl.when(k != 0)
    def _():
        o_ref[...] = o_ref[...] + p


def _dense_head(xn, W3, b3, W4, b4):
    B = xn.shape[0]
    G = W3.shape[0]
    C = W4.shape[1]
    GP = ((G + 511) // 512) * 512
    W3p = jnp.pad(W3, ((0, 0), (0, GP - G)))
    b3p = jnp.pad(b3.reshape(1, G), ((0, 0), (0, GP - G)))
    W4p = jnp.pad(W4, ((0, GP - G), (0, 0)))
    KB = 512
    nk = GP // KB
    return pl.pallas_call(
        _head_body,
        grid=(nk,),
        in_specs=[
            pl.BlockSpec((B, G), lambda k: (0, 0)),
            pl.BlockSpec((G, KB), lambda k: (0, k)),
            pl.BlockSpec((1, KB), lambda k: (0, k)),
            pl.BlockSpec((KB, C), lambda k: (k, 0)),
            pl.BlockSpec((1, C), lambda k: (0, 0)),
        ],
        out_specs=pl.BlockSpec((B, C), lambda k: (0, 0)),
        out_shape=jax.ShapeDtypeStruct((B, C), jnp.float32),
    )(xn, W3p, b3p, W4p, b4.reshape(1, C))


def kernel(x, edge_index, edge_tf, batch, W1x, b1x, W1e, b1e, c1_Wl, c1_bl,
           c1_Wr, c1_br, c1_We, c1_att, c1_bias, W2, b2, c2_Wl, c2_bl, c2_Wr,
           c2_br, c2_att, c2_bias, ln_g, ln_b, W3, b3, W4, b4):
    N = x.shape[0]
    B = N // _NUM_NODES
    n_acc = ((N + 1 + 15) // 16) * 16

    E1 = edge_index.shape[1]
    e1p = ((E1 + _NW * _EBO - 1) // (_NW * _EBO)) * (_NW * _EBO)
    srcp = jnp.pad(edge_index[0], (0, e1p - E1))
    dstp = jnp.pad(edge_index[1], (0, e1p - E1), constant_values=N)

    xh = jax.nn.sigmoid(x @ W1x + b1x)
    xl = xh @ c1_Wl + c1_bl
    xr = xh @ c1_Wr + c1_br

    xs, xd, xlg, xrg = _sc_l1_gather(srcp, dstp, x, xl, xr)
    pay1 = _tc_alpha(xs, xd, xlg, xrg, W1e, b1e, c1_We, c1_att)
    parts1 = _sc_l1_scatter(dstp, pay1, n_acc)
    nd1 = parts1[0] + parts1[1]
    x1 = ((nd1[:N, 0] / (nd1[:N, 2] + 1e-16)
           + nd1[:N, 1] / (nd1[:N, 3] + 1e-16)) / (_HEADS * _HID)
          + jnp.mean(c1_bias)).reshape(B, _NUM_NODES)

    xt = jax.nn.sigmoid(x1.reshape(N, 1) @ W2 + b2)
    yl2 = xt @ c2_Wl + c2_bl
    yr2 = xt @ c2_Wr + c2_br

    E2 = edge_tf.shape[1]
    e2p = ((E2 + _NW * _EB - 1) // (_NW * _EB)) * (_NW * _EB)
    src2p = jnp.pad(edge_tf[0], (0, e2p - E2))
    dst2p = jnp.pad(edge_tf[1], (0, e2p - E2), constant_values=N)
    att2 = c2_att.reshape(_HID)

    parts = _sc_l2(src2p, dst2p, yl2, yr2, att2, n_acc)
    nd = parts[0] + parts[1]
    num2 = nd[:N, 0]
    den2 = nd[:N, 1]
    out2_mean = num2 / (den2 + 1e-16) / _HID + jnp.mean(c2_bias)
    x2 = x1 + out2_mean.reshape(B, _NUM_NODES)

    xg = x2[:, :_NUM_GENE]
    mu = xg.mean(axis=-1, keepdims=True)
    var = xg.var(axis=-1, keepdims=True)
    xn = (xg - mu) / jnp.sqrt(var + 1e-5) * ln_g + ln_b
    return _dense_head(xn, W3, b3, W4, b4)


# speedup vs baseline: 30.6789x; 1.0875x over previous
import functools

import jax
import jax.numpy as jnp
from jax import lax
from jax.experimental import pallas as pl
from jax.experimental.pallas import tpu as pltpu
from jax.experimental.pallas import tpu_sc as plsc

_NUM_NODES = 10000
_NUM_GENE = 4000
_HID = 32
_HEADS = 2

_NC = 2
_NS = 16
_NW = _NC * _NS
_L = 16
_EB = 128
_PW = 8


_EBO = 512
_NSUB = _EBO // _EB


_CW = 80


def _sc_l1_gather_body(src_hbm, dst_hbm, cs_hbm, cd_hbm,
                       gs_out, gd_out, idx_s, idx_d, gsv, gdv, sem):
    cid = lax.axis_index("c")
    sid = lax.axis_index("s")
    wid = sid * _NC + cid
    nchunk = src_hbm.shape[0] // (_NW * _EBO)

    def chunk(g, carry):
        base = (wid * nchunk + g) * _EBO
        pltpu.sync_copy(src_hbm.at[pl.ds(base, _EBO)], idx_s)
        pltpu.sync_copy(dst_hbm.at[pl.ds(base, _EBO)], idx_d)
        cps = []
        for q in range(_NSUB):
            r = pl.ds(q * _EB, _EB)
            cps.append(pltpu.async_copy(cs_hbm.at[idx_s.at[r]], gsv.at[r], sem))
            cps.append(pltpu.async_copy(cd_hbm.at[idx_d.at[r]], gdv.at[r], sem))
        for cp in cps:
            cp.wait()
        pltpu.sync_copy(gsv, gs_out.at[pl.ds(base, _EBO)])
        pltpu.sync_copy(gdv, gd_out.at[pl.ds(base, _EBO)])
        return carry

    lax.fori_loop(0, nchunk, chunk, 0)


def _sc_l1_gather(srcp, dstp, catS, catD):
    e1p = srcp.shape[0]
    mesh = plsc.VectorSubcoreMesh(core_axis_name="c", subcore_axis_name="s")
    fn = pl.kernel(
        _sc_l1_gather_body,
        out_type=(
            jax.ShapeDtypeStruct((e1p, _CW), jnp.float32),
            jax.ShapeDtypeStruct((e1p, _CW), jnp.float32),
        ),
        mesh=mesh,
        scratch_types=[
            pltpu.VMEM((_EBO,), jnp.int32),
            pltpu.VMEM((_EBO,), jnp.int32),
            pltpu.VMEM((_EBO, _CW), jnp.float32),
            pltpu.VMEM((_EBO, _CW), jnp.float32),
            pltpu.SemaphoreType.DMA,
        ],
        compiler_params=pltpu.CompilerParams(
            needs_layout_passes=False, use_tc_tiling_on_sc=False),
    )
    return fn(srcp, dstp, catS, catD)


def _alpha_body(gs_ref, gd_ref, w1e_ref, b1e_ref, we_ref,
                att_ref, sel_ref, pay_ref):
    gs = gs_ref[...]
    gd = gd_ref[...]
    u = gs[:, :16] * gd[:, :16]
    xlg = gs[:, 16:_CW]
    xe = jax.nn.sigmoid(
        jnp.dot(u, w1e_ref[...], preferred_element_type=jnp.float32)
        + b1e_ref[...])
    ea = jnp.dot(xe, we_ref[...], preferred_element_type=jnp.float32)
    mm = (gs + gd)[:, 16:_CW] + ea
    m = jnp.maximum(mm, 0.2 * mm)
    aw = m * att_ref[...]
    alpha = jnp.dot(aw, sel_ref[...], preferred_element_type=jnp.float32)
    ex = jnp.exp(alpha)
    sl = jnp.dot(xlg, sel_ref[...], preferred_element_type=jnp.float32)
    z = jnp.zeros_like(ex)
    pay_ref[...] = jnp.concatenate([ex * sl, ex, z, z], axis=-1)


def _tc_alpha(gs, gd, W1e, b1e, c1_We, c1_att):
    e1p = gs.shape[0]
    EBK = 4096
    nk = e1p // EBK
    attv = c1_att.reshape(1, _HEADS * _HID)
    sel = jnp.repeat(jnp.eye(_HEADS, dtype=jnp.float32), _HID, axis=0)
    return pl.pallas_call(
        _alpha_body,
        grid=(nk,),
        in_specs=[
            pl.BlockSpec((EBK, _CW), lambda k: (k, 0)),
            pl.BlockSpec((EBK, _CW), lambda k: (k, 0)),
            pl.BlockSpec((16, 32), lambda k: (0, 0)),
            pl.BlockSpec((1, 32), lambda k: (0, 0)),
            pl.BlockSpec((32, 64), lambda k: (0, 0)),
            pl.BlockSpec((1, 64), lambda k: (0, 0)),
            pl.BlockSpec((64, _HEADS), lambda k: (0, 0)),
        ],
        out_specs=pl.BlockSpec((EBK, _PW), lambda k: (k, 0)),
        out_shape=jax.ShapeDtypeStruct((e1p, _PW), jnp.float32),
    )(gs, gd, W1e, b1e.reshape(1, 32), c1_We, attv, sel)


def _sc_l1_scatter_body(dst_hbm, pay_hbm, zero_hbm, out_hbm,
                        idx_d2, pay, acc):
    cid = lax.axis_index("c")
    sid = lax.axis_index("s")
    wid = sid * _NC + cid
    nchunk = dst_hbm.shape[0] // (_NW * _EBO)

    @pl.when(sid == 0)
    def _():
        pltpu.sync_copy(zero_hbm, acc)

    plsc.subcore_barrier()

    def chunk(g, carry):
        base = (wid * nchunk + g) * _EBO
        for q in range(_NSUB):
            pltpu.sync_copy(dst_hbm.at[pl.ds(base + q * _EB, _EB)],
                            idx_d2.at[q])
        pltpu.sync_copy(pay_hbm.at[pl.ds(base, _EBO)], pay)
        for q in range(_NSUB):
            pltpu.sync_copy(pay.at[pl.ds(q * _EB, _EB)],
                            acc.at[idx_d2.at[q]], add=True)
        return carry

    lax.fori_loop(0, nchunk, chunk, 0)
    plsc.subcore_barrier()

    @pl.when(sid == 0)
    def _():
        pltpu.sync_copy(acc, out_hbm.at[cid])


def _sc_l1_scatter(dstp, pay, n_acc):
    mesh = plsc.VectorSubcoreMesh(core_axis_name="c", subcore_axis_name="s")
    zero = jnp.zeros((n_acc, _PW), jnp.float32)
    fn = pl.kernel(
        _sc_l1_scatter_body,
        out_type=jax.ShapeDtypeStruct((_NC, n_acc, _PW), jnp.float32),
        mesh=mesh,
        scratch_types=[
            pltpu.VMEM((_NSUB, _EB), jnp.int32),
            pltpu.VMEM((_EBO, _PW), jnp.float32),
            pltpu.VMEM_SHARED((n_acc, _PW), jnp.float32),
        ],
        compiler_params=pltpu.CompilerParams(
            needs_layout_passes=False, use_tc_tiling_on_sc=False),
    )
    return fn(dstp, pay, zero)


def _sc_l2_body(src_hbm, dst_hbm, yl_hbm, yr_hbm, att_hbm, zero_hbm, out_hbm,
                idx_s, idx_d, ylv, yrv, pay, att_v, acc, sem1, sem2):
    cid = lax.axis_index("c")
    sid = lax.axis_index("s")
    wid = sid * _NC + cid
    nchunk = src_hbm.shape[0] // (_NW * _EB)
    iot = lax.iota(jnp.int32, _L)

    @pl.when(sid == 0)
    def _():
        pltpu.sync_copy(zero_hbm, acc)

    pltpu.sync_copy(att_hbm, att_v)
    plsc.subcore_barrier()

    def chunk(g, carry):
        base = (wid * nchunk + g) * _EB
        pltpu.sync_copy(src_hbm.at[pl.ds(base, _EB)], idx_s)
        pltpu.sync_copy(dst_hbm.at[pl.ds(base, _EB)], idx_d)
        cp1 = pltpu.async_copy(yl_hbm.at[idx_s], ylv, sem1)
        cp2 = pltpu.async_copy(yr_hbm.at[idx_d], yrv, sem2)
        cp1.wait()
        cp2.wait()

        def group(j, carry2):
            rows = j * _L + iot
            col0 = jnp.zeros((_L,), jnp.int32)

            def ch(c, ac):
                a, sl = ac
                cc = col0 + c
                ylc = plsc.load_gather(ylv, [rows, cc])
                yrc = plsc.load_gather(yrv, [rows, cc])
                attc = plsc.load_gather(att_v, [cc])
                z = ylc + yrc
                m = jnp.maximum(z, 0.2 * z)
                return (a + m * attc, sl + ylc)

            a, sl = lax.fori_loop(
                0, _HID, ch,
                (jnp.zeros((_L,), jnp.float32), jnp.zeros((_L,), jnp.float32)))
            ex = jnp.exp(a)
            plsc.store_scatter(pay, [rows, col0], ex * sl)
            plsc.store_scatter(pay, [rows, col0 + 1], ex)
            return carry2

        lax.fori_loop(0, _EB // _L, group, 0)
        pltpu.sync_copy(pay, acc.at[idx_d], add=True)
        return carry

    lax.fori_loop(0, nchunk, chunk, 0)
    plsc.subcore_barrier()

    @pl.when(sid == 0)
    def _():
        pltpu.sync_copy(acc, out_hbm.at[cid])


def _sc_l2(src2p, dst2p, yl, yr, att, n_acc):
    mesh = plsc.VectorSubcoreMesh(core_axis_name="c", subcore_axis_name="s")
    zero = jnp.zeros((n_acc, _PW), jnp.float32)
    fn = pl.kernel(
        _sc_l2_body,
        out_type=jax.ShapeDtypeStruct((_NC, n_acc, _PW), jnp.float32),
        mesh=mesh,
        scratch_types=[
            pltpu.VMEM((_EB,), jnp.int32),
            pltpu.VMEM((_EB,), jnp.int32),
            pltpu.VMEM((_EB, _HID), jnp.float32),
            pltpu.VMEM((_EB, _HID), jnp.float32),
            pltpu.VMEM((_EB, _PW), jnp.float32),
            pltpu.VMEM((_HID,), jnp.float32),
            pltpu.VMEM_SHARED((n_acc, _PW), jnp.float32),
            pltpu.SemaphoreType.DMA,
            pltpu.SemaphoreType.DMA,
        ],
        compiler_params=pltpu.CompilerParams(
            needs_layout_passes=False, use_tc_tiling_on_sc=False),
    )
    return fn(src2p, dst2p, yl, yr, att, zero)


def _head_body(xn_ref, w3_ref, b3_ref, w4_ref, b4_ref, o_ref):
    k = pl.program_id(0)
    h = jnp.dot(xn_ref[...], w3_ref[...], preferred_element_type=jnp.float32)
    h = jnp.maximum(h + b3_ref[...], 0.0)
    p = jnp.dot(h, w4_ref[...], preferred_element_type=jnp.float32)

    @pl.when(k == 0)
    def _():
        o_ref[...] = p + b4_ref[...]

    @pl.when(k != 0)
    def _():
        o_ref[...] = o_ref[...] + p


def _dense_head(xn, W3, b3, W4, b4):
    B = xn.shape[0]
    G = W3.shape[0]
    C = W4.shape[1]
    GP = ((G + 511) // 512) * 512
    W3p = jnp.pad(W3, ((0, 0), (0, GP - G)))
    b3p = jnp.pad(b3.reshape(1, G), ((0, 0), (0, GP - G)))
    W4p = jnp.pad(W4, ((0, GP - G), (0, 0)))
    KB = 512
    nk = GP // KB
    return pl.pallas_call(
        _head_body,
        grid=(nk,),
        in_specs=[
            pl.BlockSpec((B, G), lambda k: (0, 0)),
            pl.BlockSpec((G, KB), lambda k: (0, k)),
            pl.BlockSpec((1, KB), lambda k: (0, k)),
            pl.BlockSpec((KB, C), lambda k: (k, 0)),
            pl.BlockSpec((1, C), lambda k: (0, 0)),
        ],
        out_specs=pl.BlockSpec((B, C), lambda k: (0, 0)),
        out_shape=jax.ShapeDtypeStruct((B, C), jnp.float32),
    )(xn, W3p, b3p, W4p, b4.reshape(1, C))


def kernel(x, edge_index, edge_tf, batch, W1x, b1x, W1e, b1e, c1_Wl, c1_bl,
           c1_Wr, c1_br, c1_We, c1_att, c1_bias, W2, b2, c2_Wl, c2_bl, c2_Wr,
           c2_br, c2_att, c2_bias, ln_g, ln_b, W3, b3, W4, b4):
    N = x.shape[0]
    B = N // _NUM_NODES
    n_acc = ((N + 1 + 15) // 16) * 16

    E1 = edge_index.shape[1]
    e1p = ((E1 + _NW * _EBO - 1) // (_NW * _EBO)) * (_NW * _EBO)
    srcp = jnp.pad(edge_index[0], (0, e1p - E1))
    dstp = jnp.pad(edge_index[1], (0, e1p - E1), constant_values=N)

    xh = jax.nn.sigmoid(x @ W1x + b1x)
    xl = xh @ c1_Wl + c1_bl
    xr = xh @ c1_Wr + c1_br
    catS = jnp.concatenate([x, xl], axis=1)
    catD = jnp.concatenate([x, xr], axis=1)

    gs, gd = _sc_l1_gather(srcp, dstp, catS, catD)
    pay1 = _tc_alpha(gs, gd, W1e, b1e, c1_We, c1_att)
    parts1 = _sc_l1_scatter(dstp, pay1, n_acc)
    nd1 = parts1[0] + parts1[1]
    x1 = ((nd1[:N, 0] / (nd1[:N, 2] + 1e-16)
           + nd1[:N, 1] / (nd1[:N, 3] + 1e-16)) / (_HEADS * _HID)
          + jnp.mean(c1_bias)).reshape(B, _NUM_NODES)

    xt = jax.nn.sigmoid(x1.reshape(N, 1) @ W2 + b2)
    yl2 = xt @ c2_Wl + c2_bl
    yr2 = xt @ c2_Wr + c2_br

    E2 = edge_tf.shape[1]
    e2p = ((E2 + _NW * _EB - 1) // (_NW * _EB)) * (_NW * _EB)
    src2p = jnp.pad(edge_tf[0], (0, e2p - E2))
    dst2p = jnp.pad(edge_tf[1], (0, e2p - E2), constant_values=N)
    att2 = c2_att.reshape(_HID)

    parts = _sc_l2(src2p, dst2p, yl2, yr2, att2, n_acc)
    nd = parts[0] + parts[1]
    num2 = nd[:N, 0]
    den2 = nd[:N, 1]
    out2_mean = num2 / (den2 + 1e-16) / _HID + jnp.mean(c2_bias)
    x2 = x1 + out2_mean.reshape(B, _NUM_NODES)

    xg = x2[:, :_NUM_GENE]
    mu = xg.mean(axis=-1, keepdims=True)
    var = xg.var(axis=-1, keepdims=True)
    xn = (xg - mu) / jnp.sqrt(var + 1e-5) * ln_g + ln_b
    return _dense_head(xn, W3, b3, W4, b4)
